# baseline (speedup 1.0000x reference)
import functools

import jax
import jax.numpy as jnp
from jax import lax
from jax.experimental import pallas as pl
from jax.experimental.pallas import tpu as pltpu
from jax.experimental.pallas import tpu_sc as plsc

NC, NS = 2, 16
NW = NC * NS
TOK = 128
L = 16


def _prep_kernel(n_chunks, wT_hbm, out_hbm, cols_v, trows_v, isem, osem):
    wid = lax.axis_index("s") * NC + lax.axis_index("c")
    iot = lax.iota(jnp.int32, L)
    rd = [iot + d0 for d0 in range(0, 64, L)]

    def get(c, buf):
        return pltpu.make_async_copy(
            wT_hbm.at[:, pl.ds(c * TOK, TOK)], cols_v.at[buf], isem.at[buf])

    def put(c, buf):
        return pltpu.make_async_copy(
            trows_v.at[buf], out_hbm.at[pl.ds(c * TOK, TOK)], osem.at[buf])

    def chunk_of(i):
        return i * NW + wid

    my_n = lax.div(n_chunks - wid + (NW - 1), NW)

    @pl.when(my_n >= 1)
    def _():
        get(chunk_of(0), 0).start()

    @pl.when(my_n >= 2)
    def _():
        get(chunk_of(1), 1).start()

    def tbody(i, _):
        c = chunk_of(i)
        buf = lax.rem(i, 2)
        get(c, buf).wait()

        @pl.when(i >= 2)
        def _():
            put(chunk_of(i - 2), buf).wait()

        cols = cols_v.at[buf]
        trows = trows_v.at[buf]

        def kbody(k, _):
            pk = (iot + k) & (L - 1)
            for t0 in range(0, TOK, L):
                ct = pk + t0
                for j in range(64 // L):
                    v = plsc.load_gather(cols, [rd[j], ct])
                    plsc.store_scatter(trows, [ct, rd[j]], v)
            return 0

        lax.fori_loop(0, L, kbody, 0)
        put(c, buf).start()

        @pl.when(i + 2 < my_n)
        def _():
            get(chunk_of(i + 2), buf).start()

        return 0

    lax.fori_loop(0, my_n, tbody, 0)

    @pl.when(my_n >= 2)
    def _():
        put(chunk_of(my_n - 2), lax.rem(my_n - 2, 2)).wait()

    @pl.when(my_n >= 1)
    def _():
        put(chunk_of(my_n - 1), lax.rem(my_n - 1, 2)).wait()


def _emb_kernel(n_b1, idx_hbm, table_hbm, out_hbm,
                idx_v, rows_v, trows_v, gsem, osem):
    wid = lax.axis_index("s") * NC + lax.axis_index("c")
    b0w = wid * TOK

    pltpu.sync_copy(idx_hbm.at[:, wid], idx_v)

    def gather(b1, buf):
        return pltpu.make_async_copy(
            table_hbm.at[idx_v.at[b1]], rows_v.at[buf], gsem.at[buf])

    def put(b1, buf):
        return pltpu.make_async_copy(
            trows_v.at[buf], out_hbm.at[b1, :, pl.ds(b0w, TOK)],
            osem.at[buf])

    gather(0, 0).start()
    gather(1, 1).start()

    iot = lax.iota(jnp.int32, L)
    rt = [iot + t0 for t0 in range(0, TOK, L)]

    def transpose(buf):
        rows = rows_v.at[buf]
        trows = trows_v.at[buf]
        def kbody(k, _):
            pk = (iot + k) & (L - 1)
            for dblk in range(64 // L):
                cd = pk + dblk * L
                for j in range(TOK // L):
                    v = plsc.load_gather(rows, [rt[j], cd])
                    plsc.store_scatter(trows, [cd, rt[j]], v)
            return 0

        lax.fori_loop(0, L, kbody, 0)

    def body(b1, _):
        buf = lax.rem(b1, 2)
        gather(b1, buf).wait()

        @pl.when(b1 >= 2)
        def _():
            put(b1 - 2, buf).wait()

        transpose(buf)
        put(b1, buf).start()

        @pl.when(b1 + 2 < n_b1)
        def _():
            gather(b1 + 2, buf).start()

        return 0

    lax.fori_loop(0, n_b1, body, 0)
    put(n_b1 - 2, lax.rem(n_b1 - 2, 2)).wait()
    put(n_b1 - 1, lax.rem(n_b1 - 1, 2)).wait()


@jax.jit
def kernel(indices, weight):
    B0, B1 = indices.shape
    V, D = weight.shape

    idxT = indices.astype(jnp.int32).T.reshape(B1, NW, TOK)
    mesh = plsc.VectorSubcoreMesh(core_axis_name="c", subcore_axis_name="s")

    v_pad = (V + TOK - 1) // TOK * TOK
    wTp = jnp.pad(weight.T, ((0, 0), (0, v_pad - V)))
    wp = pl.kernel(
        functools.partial(_prep_kernel, v_pad // TOK),
        out_type=jax.ShapeDtypeStruct((v_pad, 128), jnp.float32),
        mesh=mesh,
        compiler_params=pltpu.CompilerParams(
            use_tc_tiling_on_sc=True, needs_layout_passes=False),
        scratch_types=[
            pltpu.VMEM((2, D, TOK), jnp.float32),
            pltpu.VMEM((2, TOK, 128), jnp.float32),
            pltpu.SemaphoreType.DMA((2,)),
            pltpu.SemaphoreType.DMA((2,)),
        ],
    )(wTp)

    out = pl.kernel(
        functools.partial(_emb_kernel, B1),
        out_type=jax.ShapeDtypeStruct((B1, D, B0), jnp.float32),
        mesh=mesh,
        compiler_params=pltpu.CompilerParams(use_tc_tiling_on_sc=True, needs_layout_passes=False),
        scratch_types=[
            pltpu.VMEM((B1, TOK), jnp.int32),
            pltpu.VMEM((2, TOK, 128), jnp.float32),
            pltpu.VMEM((2, D, TOK), jnp.float32),
            pltpu.SemaphoreType.DMA((2,)),
            pltpu.SemaphoreType.DMA((2,)),
        ],
    )(idxT, wp)
    return jnp.transpose(out, (2, 0, 1))


# speedup vs baseline: 1.0929x; 1.0929x over previous
import functools

import jax
import jax.numpy as jnp
from jax import lax
from jax.experimental import pallas as pl
from jax.experimental.pallas import tpu as pltpu
from jax.experimental.pallas import tpu_sc as plsc

NC, NS = 2, 16
NW = NC * NS
TOK = 128
L = 16


def _emb_kernel(n_b1, idx_hbm, table_hbm, out_hbm,
                idx_v, rows_v, trows_v, gsem, osem):
    wid = lax.axis_index("s") * NC + lax.axis_index("c")
    b0w = wid * TOK

    pltpu.sync_copy(idx_hbm.at[:, wid], idx_v)

    def gather(b1, buf):
        return pltpu.make_async_copy(
            table_hbm.at[idx_v.at[b1]], rows_v.at[buf], gsem.at[buf])

    def put(b1, buf):
        return pltpu.make_async_copy(
            trows_v.at[buf], out_hbm.at[b1, :, pl.ds(b0w, TOK)],
            osem.at[buf])

    gather(0, 0).start()
    gather(1, 1).start()

    iot = lax.iota(jnp.int32, L)
    rt = [iot + t0 for t0 in range(0, TOK, L)]

    def transpose(buf):
        rows = rows_v.at[buf]
        trows = trows_v.at[buf]
        def kbody(k, _):
            pk = (iot + k) & (L - 1)
            for dblk in range(64 // L):
                cd = pk + dblk * L
                for j in range(TOK // L):
                    v = plsc.load_gather(rows, [rt[j], cd])
                    plsc.store_scatter(trows, [cd, rt[j]], v)
            return 0

        lax.fori_loop(0, L, kbody, 0)

    def body(b1, _):
        buf = lax.rem(b1, 3)
        gather(b1, buf).wait()

        @pl.when(b1 + 2 < n_b1)
        def _():
            gather(b1 + 2, lax.rem(b1 + 2, 3)).start()

        @pl.when(b1 >= 3)
        def _():
            put(b1 - 3, buf).wait()

        transpose(buf)
        put(b1, buf).start()
        return 0

    lax.fori_loop(0, n_b1, body, 0)
    put(n_b1 - 3, lax.rem(n_b1 - 3, 3)).wait()
    put(n_b1 - 2, lax.rem(n_b1 - 2, 3)).wait()
    put(n_b1 - 1, lax.rem(n_b1 - 1, 3)).wait()


@jax.jit
def kernel(indices, weight):
    B0, B1 = indices.shape
    V, D = weight.shape

    wp = jnp.pad(weight, ((0, 0), (0, 128 - D)))
    idxT = indices.astype(jnp.int32).T.reshape(B1, NW, TOK)
    mesh = plsc.VectorSubcoreMesh(core_axis_name="c", subcore_axis_name="s")

    out = pl.kernel(
        functools.partial(_emb_kernel, B1),
        out_type=jax.ShapeDtypeStruct((B1, D, B0), jnp.float32),
        mesh=mesh,
        compiler_params=pltpu.CompilerParams(use_tc_tiling_on_sc=True, needs_layout_passes=False),
        scratch_types=[
            pltpu.VMEM((B1, TOK), jnp.int32),
            pltpu.VMEM((3, TOK, 128), jnp.float32),
            pltpu.VMEM((3, D, TOK), jnp.float32),
            pltpu.SemaphoreType.DMA((3,)),
            pltpu.SemaphoreType.DMA((3,)),
        ],
    )(idxT, wp)
    return jnp.transpose(out, (2, 0, 1))
